# baseline (speedup 1.0000x reference)
import jax
import jax.numpy as jnp
from jax import lax
from jax.experimental import pallas as pl
from jax.experimental.pallas import tpu as pltpu
from jax.experimental.pallas import tpu_sc as plsc

EPS = 1e-15
N = 10000
E = 160000
EPAD = 163840
F = 128
H = 4
HF = H * F
KVJ_W = 2 * HF + 3 * F
N_RBF = 20
CUTOFF = 5.0


_BN = 1000


def _k1_body(s_ref, vjc_ref, g_ref, b_ref, wq_ref, wk_ref, wv_ref,
             q_ref, kvj_ref):
    x = s_ref[...]
    m = jnp.mean(x, axis=1, keepdims=True)
    var = jnp.mean((x - m) ** 2, axis=1, keepdims=True)
    ln = (x - m) * lax.rsqrt(var + 1e-5) * g_ref[...] + b_ref[...]
    q_ref[...] = jnp.dot(ln, wq_ref[...], preferred_element_type=jnp.float32)
    kvj_ref[:, 0:HF] = jnp.dot(ln, wk_ref[...], preferred_element_type=jnp.float32)
    kvj_ref[:, HF:2 * HF] = jnp.dot(ln, wv_ref[...], preferred_element_type=jnp.float32)
    kvj_ref[:, 2 * HF:KVJ_W] = vjc_ref[...]


def _k1(s_j, vjc, ln_gamma, ln_beta, Wq, Wk, Wv):
    grid = (N // _BN,)
    return pl.pallas_call(
        _k1_body,
        grid=grid,
        in_specs=[
            pl.BlockSpec((_BN, F), lambda n: (n, 0)),
            pl.BlockSpec((_BN, 3 * F), lambda n: (n, 0)),
            pl.BlockSpec((1, F), lambda n: (0, 0)),
            pl.BlockSpec((1, F), lambda n: (0, 0)),
            pl.BlockSpec((F, HF), lambda n: (0, 0)),
            pl.BlockSpec((F, HF), lambda n: (0, 0)),
            pl.BlockSpec((F, HF), lambda n: (0, 0)),
        ],
        out_specs=[
            pl.BlockSpec((_BN, HF), lambda n: (n, 0)),
            pl.BlockSpec((_BN, KVJ_W), lambda n: (n, 0)),
        ],
        out_shape=[
            jax.ShapeDtypeStruct((N, HF), jnp.float32),
            jax.ShapeDtypeStruct((N, KVJ_W), jnp.float32),
        ],
    )(s_j, vjc, ln_gamma.reshape(1, F), ln_beta.reshape(1, F), Wq, Wk, Wv)


_NW = 32
_CHUNK = EPAD // _NW
_BT2 = 32
_NB2 = _CHUNK // _BT2


def _k2_body(q_hbm, kvj_hbm, ii_hbm, jj_hbm, gq_hbm, gkvj_hbm,
             ii0, jj0, ii1, jj1, qb0, qb1, kb0, kb1,
             sq0, sq1, sk0, sk1):
    wid = lax.axis_index("s") * 2 + lax.axis_index("c")
    base = wid * _CHUNK

    def start(blk, ii, jj, qb, kb, sq, sk):
        off = base + blk * _BT2
        pltpu.sync_copy(ii_hbm.at[pl.ds(off, _BT2)], ii)
        pltpu.sync_copy(jj_hbm.at[pl.ds(off, _BT2)], jj)
        pltpu.async_copy(q_hbm.at[ii], qb, sq)
        pltpu.async_copy(kvj_hbm.at[jj], kb, sk)

    def finish(blk, ii, jj, qb, kb, sq, sk):
        off = base + blk * _BT2
        pltpu.make_async_copy(q_hbm.at[ii], qb, sq).wait()
        pltpu.make_async_copy(kvj_hbm.at[jj], kb, sk).wait()
        pltpu.sync_copy(qb, gq_hbm.at[pl.ds(off, _BT2)])
        pltpu.sync_copy(kb, gkvj_hbm.at[pl.ds(off, _BT2)])

    start(0, ii0, jj0, qb0, kb0, sq0, sk0)

    def body(k2, _):
        ka = 2 * k2
        kb_ = ka + 1
        start(kb_, ii1, jj1, qb1, kb1, sq1, sk1)
        finish(ka, ii0, jj0, qb0, kb0, sq0, sk0)

        @pl.when(k2 + 1 < _NB2 // 2)
        def _():
            start(ka + 2, ii0, jj0, qb0, kb0, sq0, sk0)

        finish(kb_, ii1, jj1, qb1, kb1, sq1, sk1)
        return 0

    lax.fori_loop(0, _NB2 // 2, body, 0)


def _k2(Q, KVJ, idx_i, idx_j):
    mesh = plsc.VectorSubcoreMesh(core_axis_name="c", subcore_axis_name="s",
                                  num_cores=2, num_subcores=16)
    f = pl.kernel(
        _k2_body, mesh=mesh,
        out_type=[
            jax.ShapeDtypeStruct((EPAD, HF), jnp.float32),
            jax.ShapeDtypeStruct((EPAD, KVJ_W), jnp.float32),
        ],
        scratch_types=[
            pltpu.VMEM((_BT2,), jnp.int32),
            pltpu.VMEM((_BT2,), jnp.int32),
            pltpu.VMEM((_BT2,), jnp.int32),
            pltpu.VMEM((_BT2,), jnp.int32),
            pltpu.VMEM((_BT2, HF), jnp.float32),
            pltpu.VMEM((_BT2, HF), jnp.float32),
            pltpu.VMEM((_BT2, KVJ_W), jnp.float32),
            pltpu.VMEM((_BT2, KVJ_W), jnp.float32),
            pltpu.SemaphoreType.DMA,
            pltpu.SemaphoreType.DMA,
            pltpu.SemaphoreType.DMA,
            pltpu.SemaphoreType.DMA,
        ],
    )
    return f(Q, KVJ, idx_i, idx_j)


_BT3 = 640


def _k3_body(r_ref, gq_ref, gkvj_ref, wdk_ref, bdk_ref, wdv_ref, bdv_ref,
             wd_ref, bd_ref, p_ref):
    r = r_ref[...]
    dist = jnp.sqrt(jnp.sum(r * r + EPS, axis=1, keepdims=True))
    unit = r / dist
    mu = lax.broadcasted_iota(jnp.int32, (1, N_RBF), 1).astype(jnp.float32) * (
        CUTOFF / (N_RBF - 1))
    sigma = CUTOFF / (N_RBF - 1)
    rbf = jnp.exp(-((dist - mu) ** 2) / (2.0 * sigma * sigma))

    def silu(x):
        return x / (1.0 + jnp.exp(-x))

    dk = silu(jnp.dot(rbf, wdk_ref[...], preferred_element_type=jnp.float32)
              + bdk_ref[...])
    dv = silu(jnp.dot(rbf, wdv_ref[...], preferred_element_type=jnp.float32)
              + bdv_ref[...])
    q = gq_ref[...]
    k = gkvj_ref[:, 0:HF]
    v = gkvj_ref[:, HF:2 * HF]
    t = q * k * dk
    chunks = []
    for h in range(H):
        sl = slice(h * F, (h + 1) * F)
        logit = jnp.sum(t[:, sl], axis=1, keepdims=True)
        attn = silu(logit)
        chunks.append(v[:, sl] * dv[:, sl] * attn)
    msg = jnp.concatenate(chunks, axis=1)
    inv = jnp.dot(msg, wd_ref[...], preferred_element_type=jnp.float32) \
        + bd_ref[...]
    s0 = inv[:, 0:F]
    s1 = inv[:, F:2 * F]
    s2 = inv[:, 2 * F:3 * F]
    p_ref[0] = s1
    for c in range(3):
        vjc = gkvj_ref[:, 2 * HF + c * F:2 * HF + (c + 1) * F]
        p_ref[1 + c] = s0 * vjc + s2 * unit[:, c:c + 1]


def _k3(r_ij, Gq, Gkvj, Wdk, bdk, Wdv, bdv, Wd, bd):
    grid = (EPAD // _BT3,)
    return pl.pallas_call(
        _k3_body,
        grid=grid,
        in_specs=[
            pl.BlockSpec((_BT3, 3), lambda e: (e, 0)),
            pl.BlockSpec((_BT3, HF), lambda e: (e, 0)),
            pl.BlockSpec((_BT3, KVJ_W), lambda e: (e, 0)),
            pl.BlockSpec((N_RBF, HF), lambda e: (0, 0)),
            pl.BlockSpec((1, HF), lambda e: (0, 0)),
            pl.BlockSpec((N_RBF, HF), lambda e: (0, 0)),
            pl.BlockSpec((1, HF), lambda e: (0, 0)),
            pl.BlockSpec((HF, 3 * F), lambda e: (0, 0)),
            pl.BlockSpec((1, 3 * F), lambda e: (0, 0)),
        ],
        out_specs=pl.BlockSpec((4, _BT3, F), lambda e: (0, e, 0)),
        out_shape=jax.ShapeDtypeStruct((4, EPAD, F), jnp.float32),
    )(r_ij, Gq, Gkvj, Wdk, bdk.reshape(1, HF), Wdv, bdv.reshape(1, HF),
      Wd, bd.reshape(1, 3 * F))


_NT = 16
_EPT = EPAD // _NT
_BT4 = 256
_NB4 = _EPT // _BT4
_NPAD = 10240
_RPT = _NPAD // _NT


def _k4_body(p_hbm, ii_hbm, z_hbm, o_hbm, acc, idx_v, rowbuf, sem):
    cid = lax.axis_index("c")
    sid = lax.axis_index("s")
    row0 = sid * _RPT

    for rnd in range(2):
        comp = 2 * cid + rnd

        def zbody(t, _):
            rs = row0 + t * 160
            pltpu.sync_copy(z_hbm.at[pl.ds(rs, 160)], rowbuf.at[pl.ds(0, 160)])
            pltpu.sync_copy(rowbuf.at[pl.ds(0, 160)], acc.at[pl.ds(rs, 160)])
            return 0

        lax.fori_loop(0, _RPT // 160, zbody, 0)
        plsc.subcore_barrier()

        def body(k, _):
            off = sid * _EPT + k * _BT4
            pltpu.sync_copy(ii_hbm.at[pl.ds(off, _BT4)], idx_v)
            pltpu.sync_copy(p_hbm.at[comp, pl.ds(off, _BT4)], rowbuf)
            pltpu.sync_copy(rowbuf, acc.at[idx_v], add=True)
            return 0

        lax.fori_loop(0, _NB4, body, 0)
        plsc.subcore_barrier()

        def obody(t, _):
            rs = row0 + t * 160
            pltpu.sync_copy(acc.at[pl.ds(rs, 160)], rowbuf.at[pl.ds(0, 160)])
            pltpu.sync_copy(rowbuf.at[pl.ds(0, 160)],
                            o_hbm.at[comp, pl.ds(rs, 160)])
            return 0

        lax.fori_loop(0, _RPT // 160, obody, 0)
        plsc.subcore_barrier()


def _k4(P, idx_i, zeros_n):
    mesh = plsc.VectorSubcoreMesh(core_axis_name="c", subcore_axis_name="s",
                                  num_cores=2, num_subcores=16)
    f = pl.kernel(
        _k4_body, mesh=mesh,
        out_type=jax.ShapeDtypeStruct((4, _NPAD, F), jnp.float32),
        scratch_types=[
            pltpu.VMEM_SHARED((_NPAD, F), jnp.float32),
            pltpu.VMEM((_BT4,), jnp.int32),
            pltpu.VMEM((_BT4, F), jnp.float32),
            pltpu.SemaphoreType.DMA,
        ],
    )
    return f(P, idx_i, zeros_n)


def kernel(s_j, v_j, r_ij, nbrs, ln_gamma, ln_beta, Wq, Wk, Wv, Wdk, bdk,
           Wdv, bdv, Wd, bd):
    npad = EPAD - E
    idx_i = jnp.concatenate(
        [nbrs[:, 0].astype(jnp.int32), jnp.full((npad,), N, jnp.int32)])
    idx_j = jnp.concatenate(
        [nbrs[:, 1].astype(jnp.int32), jnp.zeros((npad,), jnp.int32)])
    r_pad = jnp.concatenate([r_ij, jnp.ones((npad, 3), jnp.float32)])
    vjc = jnp.transpose(v_j, (0, 2, 1)).reshape(N, 3 * F)
    Q, KVJ = _k1(s_j, vjc, ln_gamma, ln_beta, Wq, Wk, Wv)
    Gq, Gkvj = _k2(Q, KVJ, idx_i, idx_j)
    P = _k3(r_pad, Gq, Gkvj, Wdk, bdk, Wdv, bdv, Wd, bd)
    zeros_n = jnp.zeros((_NPAD, F), jnp.float32)
    O = _k4(P, idx_i, zeros_n)
    delta_s = O[0, :N]
    delta_v = jnp.stack([O[1, :N], O[2, :N], O[3, :N]], axis=-1)
    return delta_s, delta_v


# speedup vs baseline: 1.1812x; 1.1812x over previous
import functools

import jax
import jax.numpy as jnp
from jax import lax
from jax.experimental import pallas as pl
from jax.experimental.pallas import tpu as pltpu
from jax.experimental.pallas import tpu_sc as plsc

EPS = 1e-15
N = 10000
E = 160000
F = 128
H = 4
HF = H * F
KVJ_W = 2 * HF + 3 * F
N_RBF = 20
CUTOFF = 5.0


_BN = 1000


def _k1_body(s_ref, vjc_ref, g_ref, b_ref, wq_ref, wk_ref, wv_ref,
             q_ref, kvj_ref):
    x = s_ref[...]
    m = jnp.mean(x, axis=1, keepdims=True)
    var = jnp.mean((x - m) ** 2, axis=1, keepdims=True)
    ln = (x - m) * lax.rsqrt(var + 1e-5) * g_ref[...] + b_ref[...]
    q_ref[...] = jnp.dot(ln, wq_ref[...], preferred_element_type=jnp.float32)
    kvj_ref[:, 0:HF] = jnp.dot(ln, wk_ref[...], preferred_element_type=jnp.float32)
    kvj_ref[:, HF:2 * HF] = jnp.dot(ln, wv_ref[...], preferred_element_type=jnp.float32)
    kvj_ref[:, 2 * HF:KVJ_W] = vjc_ref[...]


def _k1(s_j, vjc, ln_gamma, ln_beta, Wq, Wk, Wv):
    grid = (N // _BN,)
    return pl.pallas_call(
        _k1_body,
        grid=grid,
        in_specs=[
            pl.BlockSpec((_BN, F), lambda n: (n, 0)),
            pl.BlockSpec((_BN, 3 * F), lambda n: (n, 0)),
            pl.BlockSpec((1, F), lambda n: (0, 0)),
            pl.BlockSpec((1, F), lambda n: (0, 0)),
            pl.BlockSpec((F, HF), lambda n: (0, 0)),
            pl.BlockSpec((F, HF), lambda n: (0, 0)),
            pl.BlockSpec((F, HF), lambda n: (0, 0)),
        ],
        out_specs=[
            pl.BlockSpec((_BN, HF), lambda n: (n, 0)),
            pl.BlockSpec((_BN, KVJ_W), lambda n: (n, 0)),
        ],
        out_shape=[
            jax.ShapeDtypeStruct((N, HF), jnp.float32),
            jax.ShapeDtypeStruct((N, KVJ_W), jnp.float32),
        ],
    )(s_j, vjc, ln_gamma.reshape(1, F), ln_beta.reshape(1, F), Wq, Wk, Wv)


_NW = 32
_CHUNK = E // _NW
_BT2 = 40
_NB2 = _CHUNK // _BT2


def _k2_body(q_hbm, kvj_hbm, ii_hbm, jj_hbm, gq_hbm, gkvj_hbm,
             ii_v, jj_v, qbuf, kvjbuf, sem0, sem1):
    wid = lax.axis_index("s") * 2 + lax.axis_index("c")
    base = wid * _CHUNK

    def body(k, _):
        off = base + k * _BT2
        pltpu.sync_copy(ii_hbm.at[pl.ds(off, _BT2)], ii_v)
        pltpu.sync_copy(jj_hbm.at[pl.ds(off, _BT2)], jj_v)
        cp0 = pltpu.async_copy(q_hbm.at[ii_v], qbuf, sem0)
        cp1 = pltpu.async_copy(kvj_hbm.at[jj_v], kvjbuf, sem1)
        cp0.wait()
        cp1.wait()
        pltpu.sync_copy(qbuf, gq_hbm.at[pl.ds(off, _BT2)])
        pltpu.sync_copy(kvjbuf, gkvj_hbm.at[pl.ds(off, _BT2)])
        return 0

    lax.fori_loop(0, _NB2, body, 0)


def _k2(Q, KVJ, idx_i, idx_j):
    mesh = plsc.VectorSubcoreMesh(core_axis_name="c", subcore_axis_name="s", num_cores=2, num_subcores=16)
    f = pl.kernel(
        _k2_body, mesh=mesh,
        out_type=[
            jax.ShapeDtypeStruct((E, HF), jnp.float32),
            jax.ShapeDtypeStruct((E, KVJ_W), jnp.float32),
        ],
        scratch_types=[
            pltpu.VMEM((_BT2,), jnp.int32),
            pltpu.VMEM((_BT2,), jnp.int32),
            pltpu.VMEM((_BT2, HF), jnp.float32),
            pltpu.VMEM((_BT2, KVJ_W), jnp.float32),
            pltpu.SemaphoreType.DMA,
            pltpu.SemaphoreType.DMA,
        ],
    )
    return f(Q, KVJ, idx_i, idx_j)


_BT3 = 640


def _k3_body(r_ref, gq_ref, gkvj_ref, wdk_ref, bdk_ref, wdv_ref, bdv_ref,
             wd_ref, bd_ref, p_ref):
    r = r_ref[...]
    dist = jnp.sqrt(jnp.sum(r * r + EPS, axis=1, keepdims=True))
    unit = r / dist
    mu = lax.broadcasted_iota(jnp.int32, (1, N_RBF), 1).astype(jnp.float32) * (
        CUTOFF / (N_RBF - 1))
    sigma = CUTOFF / (N_RBF - 1)
    rbf = jnp.exp(-((dist - mu) ** 2) / (2.0 * sigma * sigma))

    def silu(x):
        return x / (1.0 + jnp.exp(-x))

    dk = silu(jnp.dot(rbf, wdk_ref[...], preferred_element_type=jnp.float32)
              + bdk_ref[...])
    dv = silu(jnp.dot(rbf, wdv_ref[...], preferred_element_type=jnp.float32)
              + bdv_ref[...])
    q = gq_ref[...]
    k = gkvj_ref[:, 0:HF]
    v = gkvj_ref[:, HF:2 * HF]
    t = q * k * dk
    chunks = []
    for h in range(H):
        sl = slice(h * F, (h + 1) * F)
        logit = jnp.sum(t[:, sl], axis=1, keepdims=True)
        attn = silu(logit)
        chunks.append(v[:, sl] * dv[:, sl] * attn)
    msg = jnp.concatenate(chunks, axis=1)
    inv = jnp.dot(msg, wd_ref[...], preferred_element_type=jnp.float32) \
        + bd_ref[...]
    s0 = inv[:, 0:F]
    s1 = inv[:, F:2 * F]
    s2 = inv[:, 2 * F:3 * F]
    p_ref[0] = s1
    for c in range(3):
        vjc = gkvj_ref[:, 2 * HF + c * F:2 * HF + (c + 1) * F]
        p_ref[1 + c] = s0 * vjc + s2 * unit[:, c:c + 1]


def _k3(r_ij, Gq, Gkvj, Wdk, bdk, Wdv, bdv, Wd, bd):
    grid = (E // _BT3,)
    return pl.pallas_call(
        _k3_body,
        grid=grid,
        in_specs=[
            pl.BlockSpec((_BT3, 3), lambda e: (e, 0)),
            pl.BlockSpec((_BT3, HF), lambda e: (e, 0)),
            pl.BlockSpec((_BT3, KVJ_W), lambda e: (e, 0)),
            pl.BlockSpec((N_RBF, HF), lambda e: (0, 0)),
            pl.BlockSpec((1, HF), lambda e: (0, 0)),
            pl.BlockSpec((N_RBF, HF), lambda e: (0, 0)),
            pl.BlockSpec((1, HF), lambda e: (0, 0)),
            pl.BlockSpec((HF, 3 * F), lambda e: (0, 0)),
            pl.BlockSpec((1, 3 * F), lambda e: (0, 0)),
        ],
        out_specs=pl.BlockSpec((4, _BT3, F), lambda e: (0, e, 0)),
        out_shape=jax.ShapeDtypeStruct((4, E, F), jnp.float32),
    )(r_ij, Gq, Gkvj, Wdk, bdk.reshape(1, HF), Wdv, bdv.reshape(1, HF),
      Wd, bd.reshape(1, 3 * F))


_NT = 16
_EPT = E // _NT
_BT4 = 200
_NB4 = _EPT // _BT4
_NPAD = 10240
_RPT = _NPAD // _NT


def _k4_body(p_hbm, ii_hbm, z_hbm, o_hbm, acc, idx_v, rowbuf, sem):
    cid = lax.axis_index("c")
    sid = lax.axis_index("s")
    row0 = sid * _RPT

    for rnd in range(2):
        comp = 2 * cid + rnd
        def zbody(t, _):
            rs = row0 + t * 160
            pltpu.sync_copy(z_hbm.at[pl.ds(rs, 160)], rowbuf.at[pl.ds(0, 160)])
            pltpu.sync_copy(rowbuf.at[pl.ds(0, 160)], acc.at[pl.ds(rs, 160)])
            return 0

        lax.fori_loop(0, _RPT // 160, zbody, 0)
        plsc.subcore_barrier()

        def body(k, _):
            off = sid * _EPT + k * _BT4
            pltpu.sync_copy(ii_hbm.at[pl.ds(off, _BT4)], idx_v)
            pltpu.sync_copy(p_hbm.at[comp, pl.ds(off, _BT4)], rowbuf)
            pltpu.sync_copy(rowbuf, acc.at[idx_v], add=True)
            return 0

        lax.fori_loop(0, _NB4, body, 0)
        plsc.subcore_barrier()

        def obody(t, _):
            rs = row0 + t * 160
            pltpu.sync_copy(acc.at[pl.ds(rs, 160)], rowbuf.at[pl.ds(0, 160)])
            pltpu.sync_copy(rowbuf.at[pl.ds(0, 160)],
                            o_hbm.at[comp, pl.ds(rs, 160)])
            return 0

        lax.fori_loop(0, _RPT // 160, obody, 0)
        plsc.subcore_barrier()


def _k4(P, idx_i, zeros_n):
    mesh = plsc.VectorSubcoreMesh(core_axis_name="c", subcore_axis_name="s", num_cores=2, num_subcores=16)
    f = pl.kernel(
        _k4_body, mesh=mesh,
        out_type=jax.ShapeDtypeStruct((4, _NPAD, F), jnp.float32),
        scratch_types=[
            pltpu.VMEM_SHARED((_NPAD, F), jnp.float32),
            pltpu.VMEM((_BT4,), jnp.int32),
            pltpu.VMEM((_BT4, F), jnp.float32),
            pltpu.SemaphoreType.DMA,
        ],
    )
    return f(P, idx_i, zeros_n)


def kernel(s_j, v_j, r_ij, nbrs, ln_gamma, ln_beta, Wq, Wk, Wv, Wdk, bdk,
           Wdv, bdv, Wd, bd):
    idx_i = nbrs[:, 0].astype(jnp.int32)
    idx_j = nbrs[:, 1].astype(jnp.int32)
    vjc = jnp.transpose(v_j, (0, 2, 1)).reshape(N, 3 * F)
    Q, KVJ = _k1(s_j, vjc, ln_gamma, ln_beta, Wq, Wk, Wv)
    Gq, Gkvj = _k2(Q, KVJ, idx_i, idx_j)
    P = _k3(r_ij, Gq, Gkvj, Wdk, bdk, Wdv, bdv, Wd, bd)
    zeros_n = jnp.zeros((_NPAD, F), jnp.float32)
    O = _k4(P, idx_i, zeros_n)
    delta_s = O[0, :N]
    delta_v = jnp.stack([O[1, :N], O[2, :N], O[3, :N]], axis=-1)
    return delta_s, delta_v


# speedup vs baseline: 1.2702x; 1.0753x over previous
import functools

import jax
import jax.numpy as jnp
from jax import lax
from jax.experimental import pallas as pl
from jax.experimental.pallas import tpu as pltpu
from jax.experimental.pallas import tpu_sc as plsc

EPS = 1e-15
N = 10000
E = 160000
F = 128
H = 4
HF = H * F
KVJ_W = 2 * HF + 3 * F
N_RBF = 20
CUTOFF = 5.0


_BN = 1000


def _k1_body(s_ref, vjc_ref, g_ref, b_ref, wq_ref, wk_ref, wv_ref,
             q_ref, kvj_ref):
    x = s_ref[...]
    m = jnp.mean(x, axis=1, keepdims=True)
    var = jnp.mean((x - m) ** 2, axis=1, keepdims=True)
    ln = (x - m) * lax.rsqrt(var + 1e-5) * g_ref[...] + b_ref[...]
    q_ref[...] = jnp.dot(ln, wq_ref[...], preferred_element_type=jnp.float32)
    kvj_ref[:, 0:HF] = jnp.dot(ln, wk_ref[...], preferred_element_type=jnp.float32)
    kvj_ref[:, HF:2 * HF] = jnp.dot(ln, wv_ref[...], preferred_element_type=jnp.float32)
    kvj_ref[:, 2 * HF:KVJ_W] = vjc_ref[...]


def _k1(s_j, vjc, ln_gamma, ln_beta, Wq, Wk, Wv):
    grid = (N // _BN,)
    return pl.pallas_call(
        _k1_body,
        grid=grid,
        in_specs=[
            pl.BlockSpec((_BN, F), lambda n: (n, 0)),
            pl.BlockSpec((_BN, 3 * F), lambda n: (n, 0)),
            pl.BlockSpec((1, F), lambda n: (0, 0)),
            pl.BlockSpec((1, F), lambda n: (0, 0)),
            pl.BlockSpec((F, HF), lambda n: (0, 0)),
            pl.BlockSpec((F, HF), lambda n: (0, 0)),
            pl.BlockSpec((F, HF), lambda n: (0, 0)),
        ],
        out_specs=[
            pl.BlockSpec((_BN, HF), lambda n: (n, 0)),
            pl.BlockSpec((_BN, KVJ_W), lambda n: (n, 0)),
        ],
        out_shape=[
            jax.ShapeDtypeStruct((N, HF), jnp.float32),
            jax.ShapeDtypeStruct((N, KVJ_W), jnp.float32),
        ],
    )(s_j, vjc, ln_gamma.reshape(1, F), ln_beta.reshape(1, F), Wq, Wk, Wv)


_NW = 32
_CHUNK = E // _NW
_BT2 = 40
_NB2 = _CHUNK // _BT2


def _mk_gather_body(width):
    def body_fn(tbl_hbm, idx_hbm, out_hbm, idx_v, b0, b1, s0, s1):
        wid = lax.axis_index("s") * 2 + lax.axis_index("c")
        base = wid * _CHUNK
        pltpu.sync_copy(idx_hbm.at[pl.ds(base, _CHUNK)], idx_v)

        def start(blk, buf, sem):
            pltpu.async_copy(tbl_hbm.at[idx_v.at[pl.ds(blk * _BT2, _BT2)]],
                             buf, sem)

        def finish(blk, buf, sem):
            pltpu.make_async_copy(
                tbl_hbm.at[idx_v.at[pl.ds(blk * _BT2, _BT2)]], buf, sem).wait()
            pltpu.sync_copy(buf, out_hbm.at[pl.ds(base + blk * _BT2, _BT2)])

        start(0, b0, s0)

        def body(k2, _):
            ka = 2 * k2
            start(ka + 1, b1, s1)
            finish(ka, b0, s0)
            start(ka + 2, b0, s0)
            finish(ka + 1, b1, s1)
            return 0

        lax.fori_loop(0, _NB2 // 2, body, 0)
        finish(_NB2 - 1, b0, s0)
    return body_fn


def _gather(tbl, idx, width):
    mesh = plsc.VectorSubcoreMesh(core_axis_name="c", subcore_axis_name="s",
                                  num_cores=2, num_subcores=16)
    f = pl.kernel(
        _mk_gather_body(width), mesh=mesh,
        out_type=jax.ShapeDtypeStruct((E, width), jnp.float32),
        scratch_types=[
            pltpu.VMEM((_CHUNK,), jnp.int32),
            pltpu.VMEM((_BT2, width), jnp.float32),
            pltpu.VMEM((_BT2, width), jnp.float32),
            pltpu.SemaphoreType.DMA,
            pltpu.SemaphoreType.DMA,
        ],
    )
    return f(tbl, idx)


def _k2(Q, KVJ, idx_i, idx_j):
    Gq = _gather(Q, idx_i, HF)
    Gkvj = _gather(KVJ, idx_j, KVJ_W)
    return Gq, Gkvj


_BT3 = 640


def _k3_body(r_ref, gq_ref, gkvj_ref, wdk_ref, bdk_ref, wdv_ref, bdv_ref,
             wd_ref, bd_ref, p_ref):
    r = r_ref[...]
    dist = jnp.sqrt(jnp.sum(r * r + EPS, axis=1, keepdims=True))
    unit = r / dist
    mu = lax.broadcasted_iota(jnp.int32, (1, N_RBF), 1).astype(jnp.float32) * (
        CUTOFF / (N_RBF - 1))
    sigma = CUTOFF / (N_RBF - 1)
    rbf = jnp.exp(-((dist - mu) ** 2) / (2.0 * sigma * sigma))

    def silu(x):
        return x / (1.0 + jnp.exp(-x))

    dk = silu(jnp.dot(rbf, wdk_ref[...], preferred_element_type=jnp.float32)
              + bdk_ref[...])
    dv = silu(jnp.dot(rbf, wdv_ref[...], preferred_element_type=jnp.float32)
              + bdv_ref[...])
    q = gq_ref[...]
    k = gkvj_ref[:, 0:HF]
    v = gkvj_ref[:, HF:2 * HF]
    t = q * k * dk
    chunks = []
    for h in range(H):
        sl = slice(h * F, (h + 1) * F)
        logit = jnp.sum(t[:, sl], axis=1, keepdims=True)
        attn = silu(logit)
        chunks.append(v[:, sl] * dv[:, sl] * attn)
    msg = jnp.concatenate(chunks, axis=1)
    inv = jnp.dot(msg, wd_ref[...], preferred_element_type=jnp.float32) \
        + bd_ref[...]
    s0 = inv[:, 0:F]
    s1 = inv[:, F:2 * F]
    s2 = inv[:, 2 * F:3 * F]
    p_ref[0] = s1
    for c in range(3):
        vjc = gkvj_ref[:, 2 * HF + c * F:2 * HF + (c + 1) * F]
        p_ref[1 + c] = s0 * vjc + s2 * unit[:, c:c + 1]


def _k3(r_ij, Gq, Gkvj, Wdk, bdk, Wdv, bdv, Wd, bd):
    grid = (E // _BT3,)
    return pl.pallas_call(
        _k3_body,
        grid=grid,
        in_specs=[
            pl.BlockSpec((_BT3, 3), lambda e: (e, 0)),
            pl.BlockSpec((_BT3, HF), lambda e: (e, 0)),
            pl.BlockSpec((_BT3, KVJ_W), lambda e: (e, 0)),
            pl.BlockSpec((N_RBF, HF), lambda e: (0, 0)),
            pl.BlockSpec((1, HF), lambda e: (0, 0)),
            pl.BlockSpec((N_RBF, HF), lambda e: (0, 0)),
            pl.BlockSpec((1, HF), lambda e: (0, 0)),
            pl.BlockSpec((HF, 3 * F), lambda e: (0, 0)),
            pl.BlockSpec((1, 3 * F), lambda e: (0, 0)),
        ],
        out_specs=pl.BlockSpec((4, _BT3, F), lambda e: (0, e, 0)),
        out_shape=jax.ShapeDtypeStruct((4, E, F), jnp.float32),
    )(r_ij, Gq, Gkvj, Wdk, bdk.reshape(1, HF), Wdv, bdv.reshape(1, HF),
      Wd, bd.reshape(1, 3 * F))


_NT = 16
_EPT = E // _NT
_BT4 = 200
_NB4 = _EPT // _BT4
_NPAD = 10240
_RPT = _NPAD // _NT


def _k4_body(p_hbm, ii_hbm, z_hbm, o_hbm, acc, idx_v, rowbuf, sem):
    cid = lax.axis_index("c")
    sid = lax.axis_index("s")
    row0 = sid * _RPT

    for rnd in range(2):
        comp = 2 * cid + rnd
        def zbody(t, _):
            rs = row0 + t * 160
            pltpu.sync_copy(z_hbm.at[pl.ds(rs, 160)], rowbuf.at[pl.ds(0, 160)])
            pltpu.sync_copy(rowbuf.at[pl.ds(0, 160)], acc.at[pl.ds(rs, 160)])
            return 0

        lax.fori_loop(0, _RPT // 160, zbody, 0)
        plsc.subcore_barrier()

        def body(k, _):
            off = sid * _EPT + k * _BT4
            pltpu.sync_copy(ii_hbm.at[pl.ds(off, _BT4)], idx_v)
            pltpu.sync_copy(p_hbm.at[comp, pl.ds(off, _BT4)], rowbuf)
            pltpu.sync_copy(rowbuf, acc.at[idx_v], add=True)
            return 0

        lax.fori_loop(0, _NB4, body, 0)
        plsc.subcore_barrier()

        def obody(t, _):
            rs = row0 + t * 160
            pltpu.sync_copy(acc.at[pl.ds(rs, 160)], rowbuf.at[pl.ds(0, 160)])
            pltpu.sync_copy(rowbuf.at[pl.ds(0, 160)],
                            o_hbm.at[comp, pl.ds(rs, 160)])
            return 0

        lax.fori_loop(0, _RPT // 160, obody, 0)
        plsc.subcore_barrier()


def _k4(P, idx_i, zeros_n):
    mesh = plsc.VectorSubcoreMesh(core_axis_name="c", subcore_axis_name="s", num_cores=2, num_subcores=16)
    f = pl.kernel(
        _k4_body, mesh=mesh,
        out_type=jax.ShapeDtypeStruct((4, _NPAD, F), jnp.float32),
        scratch_types=[
            pltpu.VMEM_SHARED((_NPAD, F), jnp.float32),
            pltpu.VMEM((_BT4,), jnp.int32),
            pltpu.VMEM((_BT4, F), jnp.float32),
            pltpu.SemaphoreType.DMA,
        ],
    )
    return f(P, idx_i, zeros_n)


def kernel(s_j, v_j, r_ij, nbrs, ln_gamma, ln_beta, Wq, Wk, Wv, Wdk, bdk,
           Wdv, bdv, Wd, bd):
    idx_i = nbrs[:, 0].astype(jnp.int32)
    idx_j = nbrs[:, 1].astype(jnp.int32)
    vjc = jnp.transpose(v_j, (0, 2, 1)).reshape(N, 3 * F)
    Q, KVJ = _k1(s_j, vjc, ln_gamma, ln_beta, Wq, Wk, Wv)
    Gq, Gkvj = _k2(Q, KVJ, idx_i, idx_j)
    P = _k3(r_ij, Gq, Gkvj, Wdk, bdk, Wdv, bdv, Wd, bd)
    zeros_n = jnp.zeros((_NPAD, F), jnp.float32)
    O = _k4(P, idx_i, zeros_n)
    delta_s = O[0, :N]
    delta_v = jnp.stack([O[1, :N], O[2, :N], O[3, :N]], axis=-1)
    return delta_s, delta_v


# speedup vs baseline: 1.3104x; 1.0317x over previous
import functools

import jax
import jax.numpy as jnp
from jax import lax
from jax.experimental import pallas as pl
from jax.experimental.pallas import tpu as pltpu
from jax.experimental.pallas import tpu_sc as plsc

EPS = 1e-15
N = 10000
E = 160000
F = 128
H = 4
HF = H * F
KVJ_W = 2 * HF + 3 * F
N_RBF = 20
CUTOFF = 5.0


_BN = 1000


def _k1_body(s_ref, vjc_ref, g_ref, b_ref, wq_ref, wk_ref, wv_ref,
             q_ref, kvj_ref):
    x = s_ref[...]
    m = jnp.mean(x, axis=1, keepdims=True)
    var = jnp.mean((x - m) ** 2, axis=1, keepdims=True)
    ln = (x - m) * lax.rsqrt(var + 1e-5) * g_ref[...] + b_ref[...]
    q_ref[...] = jnp.dot(ln, wq_ref[...], preferred_element_type=jnp.float32)
    kvj_ref[:, 0:HF] = jnp.dot(ln, wk_ref[...], preferred_element_type=jnp.float32)
    kvj_ref[:, HF:2 * HF] = jnp.dot(ln, wv_ref[...], preferred_element_type=jnp.float32)
    kvj_ref[:, 2 * HF:KVJ_W] = vjc_ref[...]


def _k1(s_j, vjc, ln_gamma, ln_beta, Wq, Wk, Wv):
    grid = (N // _BN,)
    return pl.pallas_call(
        _k1_body,
        grid=grid,
        in_specs=[
            pl.BlockSpec((_BN, F), lambda n: (n, 0)),
            pl.BlockSpec((_BN, 3 * F), lambda n: (n, 0)),
            pl.BlockSpec((1, F), lambda n: (0, 0)),
            pl.BlockSpec((1, F), lambda n: (0, 0)),
            pl.BlockSpec((F, HF), lambda n: (0, 0)),
            pl.BlockSpec((F, HF), lambda n: (0, 0)),
            pl.BlockSpec((F, HF), lambda n: (0, 0)),
        ],
        out_specs=[
            pl.BlockSpec((_BN, HF), lambda n: (n, 0)),
            pl.BlockSpec((_BN, KVJ_W), lambda n: (n, 0)),
        ],
        out_shape=[
            jax.ShapeDtypeStruct((N, HF), jnp.float32),
            jax.ShapeDtypeStruct((N, KVJ_W), jnp.float32),
        ],
    )(s_j, vjc, ln_gamma.reshape(1, F), ln_beta.reshape(1, F), Wq, Wk, Wv)


_NW = 32
_CHUNK = E // _NW
_BT2 = 40
_NB2 = _CHUNK // _BT2


def _mk_gather_body(width):
    def body_fn(tbl_hbm, idx_hbm, out_hbm, idx_v, b0, b1, s0, s1):
        wid = lax.axis_index("s") * 2 + lax.axis_index("c")
        base = wid * _CHUNK
        pltpu.sync_copy(idx_hbm.at[pl.ds(base, _CHUNK)], idx_v)

        def start(blk, buf, sem):
            pltpu.async_copy(tbl_hbm.at[idx_v.at[pl.ds(blk * _BT2, _BT2)]],
                             buf, sem)

        def finish(blk, buf, sem):
            pltpu.make_async_copy(
                tbl_hbm.at[idx_v.at[pl.ds(blk * _BT2, _BT2)]], buf, sem).wait()
            pltpu.sync_copy(buf, out_hbm.at[pl.ds(base + blk * _BT2, _BT2)])

        start(0, b0, s0)

        def body(k2, _):
            ka = 2 * k2
            start(ka + 1, b1, s1)
            finish(ka, b0, s0)
            start(ka + 2, b0, s0)
            finish(ka + 1, b1, s1)
            return 0

        lax.fori_loop(0, _NB2 // 2, body, 0)
        finish(_NB2 - 1, b0, s0)
    return body_fn


def _gather(tbl, idx, width):
    mesh = plsc.VectorSubcoreMesh(core_axis_name="c", subcore_axis_name="s",
                                  num_cores=2, num_subcores=16)
    f = pl.kernel(
        _mk_gather_body(width), mesh=mesh,
        out_type=jax.ShapeDtypeStruct((E, width), jnp.float32),
        scratch_types=[
            pltpu.VMEM((_CHUNK,), jnp.int32),
            pltpu.VMEM((_BT2, width), jnp.float32),
            pltpu.VMEM((_BT2, width), jnp.float32),
            pltpu.SemaphoreType.DMA,
            pltpu.SemaphoreType.DMA,
        ],
    )
    return f(tbl, idx)


def _k2(Q, KVJ, idx_i, idx_j):
    Gq = _gather(Q, idx_i, HF)
    Gkvj = _gather(KVJ, idx_j, KVJ_W)
    return Gq, Gkvj


_BT3 = 640


def _k3_body(r_ref, gq_ref, gkvj_ref, wdk_ref, bdk_ref, wdv_ref, bdv_ref,
             wd_ref, bd_ref, p_ref):
    r = r_ref[...]
    dist = jnp.sqrt(jnp.sum(r * r + EPS, axis=1, keepdims=True))
    unit = r / dist
    mu = lax.broadcasted_iota(jnp.int32, (1, N_RBF), 1).astype(jnp.float32) * (
        CUTOFF / (N_RBF - 1))
    sigma = CUTOFF / (N_RBF - 1)
    rbf = jnp.exp(-((dist - mu) ** 2) / (2.0 * sigma * sigma))

    def silu(x):
        return x / (1.0 + jnp.exp(-x))

    dk = silu(jnp.dot(rbf, wdk_ref[...], preferred_element_type=jnp.float32)
              + bdk_ref[...])
    dv = silu(jnp.dot(rbf, wdv_ref[...], preferred_element_type=jnp.float32)
              + bdv_ref[...])
    q = gq_ref[...]
    k = gkvj_ref[:, 0:HF]
    v = gkvj_ref[:, HF:2 * HF]
    t = q * k * dk
    chunks = []
    for h in range(H):
        sl = slice(h * F, (h + 1) * F)
        logit = jnp.sum(t[:, sl], axis=1, keepdims=True)
        attn = silu(logit)
        chunks.append(v[:, sl] * dv[:, sl] * attn)
    msg = jnp.concatenate(chunks, axis=1)
    inv = jnp.dot(msg, wd_ref[...], preferred_element_type=jnp.float32) \
        + bd_ref[...]
    s0 = inv[:, 0:F]
    s1 = inv[:, F:2 * F]
    s2 = inv[:, 2 * F:3 * F]
    p_ref[0] = s1
    for c in range(3):
        vjc = gkvj_ref[:, 2 * HF + c * F:2 * HF + (c + 1) * F]
        p_ref[1 + c] = s0 * vjc + s2 * unit[:, c:c + 1]


def _k3(r_ij, Gq, Gkvj, Wdk, bdk, Wdv, bdv, Wd, bd):
    grid = (E // _BT3,)
    return pl.pallas_call(
        _k3_body,
        grid=grid,
        in_specs=[
            pl.BlockSpec((_BT3, 3), lambda e: (e, 0)),
            pl.BlockSpec((_BT3, HF), lambda e: (e, 0)),
            pl.BlockSpec((_BT3, KVJ_W), lambda e: (e, 0)),
            pl.BlockSpec((N_RBF, HF), lambda e: (0, 0)),
            pl.BlockSpec((1, HF), lambda e: (0, 0)),
            pl.BlockSpec((N_RBF, HF), lambda e: (0, 0)),
            pl.BlockSpec((1, HF), lambda e: (0, 0)),
            pl.BlockSpec((HF, 3 * F), lambda e: (0, 0)),
            pl.BlockSpec((1, 3 * F), lambda e: (0, 0)),
        ],
        out_specs=pl.BlockSpec((4, _BT3, F), lambda e: (0, e, 0)),
        out_shape=jax.ShapeDtypeStruct((4, E, F), jnp.float32),
    )(r_ij, Gq, Gkvj, Wdk, bdk.reshape(1, HF), Wdv, bdv.reshape(1, HF),
      Wd, bd.reshape(1, 3 * F))


_NT = 16
_EPT = E // _NT
_BT4 = 80
_NB4 = _EPT // _BT4
_NPAD = 10240
_RPT = _NPAD // _NT


def _k4_body(p_hbm, ii_hbm, z_hbm, o_hbm, acc, ib0, ib1, rb0, rb1, s0, s1):
    cid = lax.axis_index("c")
    sid = lax.axis_index("s")
    row0 = sid * _RPT

    for rnd in range(2):
        comp = 2 * cid + rnd

        def zbody(t, _):
            rs = row0 + t * 80
            pltpu.sync_copy(z_hbm.at[pl.ds(rs, 80)], rb0)
            pltpu.sync_copy(rb0, acc.at[pl.ds(rs, 80)])
            return 0

        lax.fori_loop(0, _RPT // 80, zbody, 0)
        plsc.subcore_barrier()

        def start(blk, ib, rb, sem):
            off = sid * _EPT + blk * _BT4
            pltpu.sync_copy(ii_hbm.at[pl.ds(off, _BT4)], ib)
            pltpu.async_copy(p_hbm.at[comp, pl.ds(off, _BT4)], rb, sem)

        def finish(blk, ib, rb, sem):
            off = sid * _EPT + blk * _BT4
            pltpu.make_async_copy(p_hbm.at[comp, pl.ds(off, _BT4)], rb,
                                  sem).wait()
            pltpu.sync_copy(rb, acc.at[ib], add=True)

        start(0, ib0, rb0, s0)

        def body(k2, _):
            ka = 2 * k2
            start(ka + 1, ib1, rb1, s1)
            finish(ka, ib0, rb0, s0)
            start(ka + 2, ib0, rb0, s0)
            finish(ka + 1, ib1, rb1, s1)
            return 0

        lax.fori_loop(0, _NB4 // 2, body, 0)
        finish(_NB4 - 1, ib0, rb0, s0)
        plsc.subcore_barrier()

        def obody(t, _):
            rs = row0 + t * 80
            pltpu.sync_copy(acc.at[pl.ds(rs, 80)], rb0)
            pltpu.sync_copy(rb0, o_hbm.at[comp, pl.ds(rs, 80)])
            return 0

        lax.fori_loop(0, _RPT // 80, obody, 0)
        plsc.subcore_barrier()


def _k4(P, idx_i, zeros_n):
    mesh = plsc.VectorSubcoreMesh(core_axis_name="c", subcore_axis_name="s",
                                  num_cores=2, num_subcores=16)
    f = pl.kernel(
        _k4_body, mesh=mesh,
        out_type=jax.ShapeDtypeStruct((4, _NPAD, F), jnp.float32),
        scratch_types=[
            pltpu.VMEM_SHARED((_NPAD, F), jnp.float32),
            pltpu.VMEM((_BT4,), jnp.int32),
            pltpu.VMEM((_BT4,), jnp.int32),
            pltpu.VMEM((_BT4, F), jnp.float32),
            pltpu.VMEM((_BT4, F), jnp.float32),
            pltpu.SemaphoreType.DMA,
            pltpu.SemaphoreType.DMA,
        ],
    )
    return f(P, idx_i, zeros_n)


def kernel(s_j, v_j, r_ij, nbrs, ln_gamma, ln_beta, Wq, Wk, Wv, Wdk, bdk,
           Wdv, bdv, Wd, bd):
    idx_i = nbrs[:, 0].astype(jnp.int32)
    idx_j = nbrs[:, 1].astype(jnp.int32)
    vjc = jnp.transpose(v_j, (0, 2, 1)).reshape(N, 3 * F)
    Q, KVJ = _k1(s_j, vjc, ln_gamma, ln_beta, Wq, Wk, Wv)
    Gq, Gkvj = _k2(Q, KVJ, idx_i, idx_j)
    P = _k3(r_ij, Gq, Gkvj, Wdk, bdk, Wdv, bdv, Wd, bd)
    zeros_n = jnp.zeros((_NPAD, F), jnp.float32)
    O = _k4(P, idx_i, zeros_n)
    delta_s = O[0, :N]
    delta_v = jnp.stack([O[1, :N], O[2, :N], O[3, :N]], axis=-1)
    return delta_s, delta_v
